# Initial kernel scaffold; baseline (speedup 1.0000x reference)
import jax
import jax.numpy as jnp
from jax.experimental import pallas as pl


def kernel(x, conv1_w, conv1_b, w2, b2, w3, b3, fc1_w, fc1_b, fc2_w, fc2_b, node_index, perm, L0, L2):
    raise NotImplementedError("write your pallas kernel here")


# speedup vs baseline: 1.2420x; 1.2420x over previous
import functools
import jax
import jax.numpy as jnp
from jax import lax
from jax.experimental import pallas as pl
from jax.experimental.pallas import tpu as pltpu
from jax.experimental.pallas import tpu_sc as plsc

B = 16
CL1_F = 32
CL2_F = 64
CL3_F = 128
IN_V = 4096
V_SEL = 4000
GRID = 64
V2 = 1024
V3 = 256
K = 4
FC1_IN = CL3_F * V3
FC1_F = 512
FC2_F = 10
VPAD = 4104


def _conv_body(x_ref, wm_ref, b_ref, out_ref):
    wm = wm_ref[...]
    bias = b_ref[...]
    xb = x_ref[0]
    xp = jnp.pad(xb, ((0, 0), (2, 2), (2, 2)))
    taps = []
    for dy in range(5):
        for dx in range(5):
            taps.append(xp[:, dy:dy + 128, dx:dx + 128].reshape(3, 128 * 128))
    p = jnp.concatenate(taps, axis=0)
    h = lax.dot_general(p, wm, (((0,), (1,)), ((), ())),
                        preferred_element_type=jnp.float32)
    h = jnp.maximum(h + bias, 0.0)
    h = h.reshape(8192, 2, CL1_F).max(axis=1)
    h = h.reshape(64, 2, 64, CL1_F).max(axis=1)
    out_ref[0] = h.reshape(IN_V, CL1_F)


def _conv_pool(x, wm, b):
    return pl.pallas_call(
        _conv_body,
        grid=(B,),
        in_specs=[
            pl.BlockSpec((1, 3, 128, 128), lambda i: (i, 0, 0, 0)),
            pl.BlockSpec((CL1_F, 75), lambda i: (0, 0)),
            pl.BlockSpec((1, CL1_F), lambda i: (0, 0)),
        ],
        out_specs=pl.BlockSpec((1, IN_V, CL1_F), lambda i: (i, 0, 0)),
        out_shape=jax.ShapeDtypeStruct((B, IN_V, CL1_F), jnp.float32),
    )(x, wm, b)


def _sc_gather(table, idx):
    info = plsc.get_sparse_core_info()
    nw = info.num_cores * info.num_subcores
    b_per_w = IN_V // nw
    mesh = plsc.VectorSubcoreMesh(core_axis_name="c", subcore_axis_name="s")

    @functools.partial(
        pl.kernel, mesh=mesh,
        out_type=jax.ShapeDtypeStruct((IN_V, B * CL1_F), jnp.float32),
        scratch_types=[
            pltpu.VMEM((b_per_w,), jnp.int32),
            pltpu.VMEM((b_per_w, B * CL1_F), jnp.float32),
            pltpu.SemaphoreType.DMA,
        ],
    )
    def k(table_hbm, idx_hbm, out_hbm, idx_v, rows_v, sem):
        wid = lax.axis_index("s") * info.num_cores + lax.axis_index("c")
        base = wid * b_per_w
        pltpu.sync_copy(idx_hbm.at[pl.ds(base, b_per_w)], idx_v)
        pltpu.async_copy(table_hbm.at[idx_v], rows_v, sem).wait()
        pltpu.sync_copy(rows_v, out_hbm.at[pl.ds(base, b_per_w)])

    return k(table, idx)


def _bn_body(x_ref, s_ref, out_ref):
    x = x_ref[...]
    s = s_ref[...]
    n = float(IN_V * B)
    csum = jnp.sum(x, axis=0, keepdims=True)
    csum2 = jnp.sum(x * x, axis=0, keepdims=True)
    chs = lax.dot_general(csum, s, (((1,), (0,)), ((), ())),
                          preferred_element_type=jnp.float32)
    chs2 = lax.dot_general(csum2, s, (((1,), (0,)), ((), ())),
                           preferred_element_type=jnp.float32)
    mean = chs / n
    var = chs2 / n - mean * mean
    inv = lax.rsqrt(var + 1e-5)
    mean_c = lax.dot_general(mean, s, (((1,), (1,)), ((), ())),
                             preferred_element_type=jnp.float32)
    inv_c = lax.dot_general(inv, s, (((1,), (1,)), ((), ())),
                            preferred_element_type=jnp.float32)
    out_ref[...] = (x - mean_c) * inv_c


def _bn(x, s):
    return pl.pallas_call(
        _bn_body,
        out_shape=jax.ShapeDtypeStruct(x.shape, jnp.float32),
    )(x, s)


def _cheb_body(alpha, beta, l_ref, xin_ref, xprev_ref, out_ref):
    acc = jnp.dot(l_ref[...], xin_ref[...], preferred_element_type=jnp.float32)
    out_ref[...] = alpha * acc + beta * xprev_ref[...]


def _cheb(l_mat, xin, xprev, alpha, beta, tile):
    v, c = l_mat.shape[0], xin.shape[1]
    grid = v // tile
    return pl.pallas_call(
        functools.partial(_cheb_body, alpha, beta),
        grid=(grid,),
        in_specs=[
            pl.BlockSpec((tile, v), lambda i: (i, 0)),
            pl.BlockSpec((v, c), lambda i: (0, 0)),
            pl.BlockSpec((tile, c), lambda i: (i, 0)),
        ],
        out_specs=pl.BlockSpec((tile, c), lambda i: (i, 0)),
        out_shape=jax.ShapeDtypeStruct((v, c), jnp.float32),
    )(l_mat, xin, xprev)


def _comb_body(nb, cin, cout, x0_ref, x1_ref, x2_ref, x3_ref, w_ref, b_ref, out_ref):
    w = w_ref[...]
    bias = b_ref[...]
    v = x0_ref.shape[0]
    outs = []
    for j in range(nb):
        sl = slice(j * cin, (j + 1) * cin)
        y = lax.dot_general(x0_ref[:, sl], w[0], (((1,), (0,)), ((), ())),
                            preferred_element_type=jnp.float32)
        y += lax.dot_general(x1_ref[:, sl], w[1], (((1,), (0,)), ((), ())),
                             preferred_element_type=jnp.float32)
        y += lax.dot_general(x2_ref[:, sl], w[2], (((1,), (0,)), ((), ())),
                             preferred_element_type=jnp.float32)
        y += lax.dot_general(x3_ref[:, sl], w[3], (((1,), (0,)), ((), ())),
                             preferred_element_type=jnp.float32)
        y = jnp.maximum(y + bias, 0.0)
        y = y.reshape(v // 4, 4, cout).max(axis=1)
        outs.append(y)
    out_ref[...] = jnp.concatenate(outs, axis=1)


def _combine(x0, x1, x2, x3, w, b, nb, cin, cout):
    v = x0.shape[0]
    grid = B // nb
    blk_in = nb * cin
    blk_out = nb * cout
    return pl.pallas_call(
        functools.partial(_comb_body, nb, cin, cout),
        grid=(grid,),
        in_specs=[
            pl.BlockSpec((v, blk_in), lambda i: (0, i)),
            pl.BlockSpec((v, blk_in), lambda i: (0, i)),
            pl.BlockSpec((v, blk_in), lambda i: (0, i)),
            pl.BlockSpec((v, blk_in), lambda i: (0, i)),
            pl.BlockSpec((K, cin, cout), lambda i: (0, 0, 0)),
            pl.BlockSpec((1, cout), lambda i: (0, 0)),
        ],
        out_specs=pl.BlockSpec((v // 4, blk_out), lambda i: (0, i)),
        out_shape=jax.ShapeDtypeStruct((v // 4, B * cout), jnp.float32),
    )(x0, x1, x2, x3, w, b)


def _comb2_body(x0_ref, x1_ref, x2_ref, x3_ref, w_ref, b_ref, out_ref):
    w = w_ref[...]
    bias = b_ref[...]
    for j in range(2):
        sl = slice(j * CL2_F, (j + 1) * CL2_F)
        y = lax.dot_general(x0_ref[:, sl], w[0], (((1,), (0,)), ((), ())),
                            preferred_element_type=jnp.float32)
        y += lax.dot_general(x1_ref[:, sl], w[1], (((1,), (0,)), ((), ())),
                             preferred_element_type=jnp.float32)
        y += lax.dot_general(x2_ref[:, sl], w[2], (((1,), (0,)), ((), ())),
                             preferred_element_type=jnp.float32)
        y += lax.dot_general(x3_ref[:, sl], w[3], (((1,), (0,)), ((), ())),
                             preferred_element_type=jnp.float32)
        y = jnp.maximum(y + bias, 0.0)
        y = y.reshape(V3, 4, CL3_F).max(axis=1)
        out_ref[j] = y.T


def _combine2(x0, x1, x2, x3, w, b):
    return pl.pallas_call(
        _comb2_body,
        grid=(8,),
        in_specs=[
            pl.BlockSpec((V2, 2 * CL2_F), lambda i: (0, i)),
            pl.BlockSpec((V2, 2 * CL2_F), lambda i: (0, i)),
            pl.BlockSpec((V2, 2 * CL2_F), lambda i: (0, i)),
            pl.BlockSpec((V2, 2 * CL2_F), lambda i: (0, i)),
            pl.BlockSpec((K, CL2_F, CL3_F), lambda i: (0, 0, 0)),
            pl.BlockSpec((1, CL3_F), lambda i: (0, 0)),
        ],
        out_specs=pl.BlockSpec((2, CL3_F, V3), lambda i: (i, 0, 0)),
        out_shape=jax.ShapeDtypeStruct((B, CL3_F, V3), jnp.float32),
    )(x0, x1, x2, x3, w, b)


def _fc_body(h_ref, w1_ref, b1_ref, w2_ref, b2_ref, out_ref, acc_ref):
    i = pl.program_id(0)

    @pl.when(i == 0)
    def _():
        acc_ref[...] = jnp.zeros_like(acc_ref)

    acc_ref[...] += lax.dot_general(h_ref[...], w1_ref[...],
                                    (((1,), (1,)), ((), ())),
                                    preferred_element_type=jnp.float32)

    @pl.when(i == pl.num_programs(0) - 1)
    def _():
        a = jnp.maximum(acc_ref[...] + b1_ref[...], 0.0)
        out_ref[...] = lax.dot_general(a, w2_ref[...], (((1,), (1,)), ((), ())),
                                       preferred_element_type=jnp.float32) + b2_ref[...]


def _fc(h, w1, b1, w2, b2):
    kchunk = 4096
    grid = FC1_IN // kchunk
    return pl.pallas_call(
        _fc_body,
        grid=(grid,),
        in_specs=[
            pl.BlockSpec((B, kchunk), lambda i: (0, i)),
            pl.BlockSpec((FC1_F, kchunk), lambda i: (0, i)),
            pl.BlockSpec((1, FC1_F), lambda i: (0, 0)),
            pl.BlockSpec((FC2_F, FC1_F), lambda i: (0, 0)),
            pl.BlockSpec((1, FC2_F), lambda i: (0, 0)),
        ],
        out_specs=pl.BlockSpec((B, FC2_F), lambda i: (0, 0)),
        out_shape=jax.ShapeDtypeStruct((B, FC2_F), jnp.float32),
        scratch_shapes=[pltpu.VMEM((B, FC1_F), jnp.float32)],
    )(h, w1, b1, w2, b2)


def kernel(x, conv1_w, conv1_b, w2, b2, w3, b3, fc1_w, fc1_b, fc2_w, fc2_b,
           node_index, perm, L0, L2):
    wm = conv1_w.transpose(0, 2, 3, 1).reshape(CL1_F, 75)
    idxflat = (node_index[:, 0] * GRID + node_index[:, 1]).astype(jnp.int32)
    perm = perm.astype(jnp.int32)
    src = jnp.where(perm < V_SEL,
                    idxflat[jnp.clip(perm, 0, V_SEL - 1)],
                    jnp.int32(IN_V)).astype(jnp.int32)
    sel = (jnp.arange(B * CL1_F, dtype=jnp.int32) % CL1_F)[:, None] == \
        jnp.arange(CL1_F, dtype=jnp.int32)[None, :]
    sel = sel.astype(jnp.float32)
    w2s = w2.reshape(CL2_F, CL1_F, K).transpose(2, 1, 0)
    w3s = w3.reshape(CL3_F, CL2_F, K).transpose(2, 1, 0)

    pooled = _conv_pool(x, wm, conv1_b.reshape(1, CL1_F))
    table = pooled.transpose(1, 0, 2).reshape(IN_V, B * CL1_F)
    table = jnp.concatenate(
        [table, jnp.zeros((VPAD - IN_V, B * CL1_F), jnp.float32)], axis=0)
    x0 = _sc_gather(table, src)
    x0 = _bn(x0, sel)
    x1 = _cheb(L0, x0, x0, 1.0, 0.0, 512)
    x2 = _cheb(L0, x1, x0, 2.0, -1.0, 512)
    x3 = _cheb(L0, x2, x1, 2.0, -1.0, 512)
    g2 = _combine(x0, x1, x2, x3, w2s, b2.reshape(1, CL2_F),
                  4, CL1_F, CL2_F)
    y1 = _cheb(L2, g2, g2, 1.0, 0.0, 512)
    y2 = _cheb(L2, y1, g2, 2.0, -1.0, 512)
    y3 = _cheb(L2, y2, y1, 2.0, -1.0, 512)
    h3 = _combine2(g2, y1, y2, y3, w3s, b3.reshape(1, CL3_F))
    hf = h3.reshape(B, FC1_IN)
    return _fc(hf, fc1_w, fc1_b.reshape(1, FC1_F),
               fc2_w, fc2_b.reshape(1, FC2_F))


# speedup vs baseline: 1.2634x; 1.0172x over previous
import functools
import jax
import jax.numpy as jnp
from jax import lax
from jax.experimental import pallas as pl
from jax.experimental.pallas import tpu as pltpu
from jax.experimental.pallas import tpu_sc as plsc

B = 16
CL1_F = 32
CL2_F = 64
CL3_F = 128
IN_V = 4096
V_SEL = 4000
GRID = 64
V2 = 1024
V3 = 256
K = 4
FC1_IN = CL3_F * V3
FC1_F = 512
FC2_F = 10
VPAD = 4104


def _conv_body(x_ref, wm_ref, b_ref, out_ref):
    wm = wm_ref[...]
    bias = b_ref[...]
    xb = x_ref[0]
    xp = jnp.pad(xb, ((0, 0), (2, 2), (2, 2)))
    taps = []
    for dy in range(5):
        for dx in range(5):
            taps.append(xp[:, dy:dy + 128, dx:dx + 128].reshape(3, 128 * 128))
    p = jnp.concatenate(taps, axis=0)
    h = lax.dot_general(p, wm, (((0,), (1,)), ((), ())),
                        preferred_element_type=jnp.float32)
    h = jnp.maximum(h + bias, 0.0)
    h = h.reshape(8192, 2, CL1_F).max(axis=1)
    h = h.reshape(64, 2, 64, CL1_F).max(axis=1)
    out_ref[0] = h.reshape(IN_V, CL1_F)


def _conv_pool(x, wm, b):
    return pl.pallas_call(
        _conv_body,
        grid=(B,),
        in_specs=[
            pl.BlockSpec((1, 3, 128, 128), lambda i: (i, 0, 0, 0)),
            pl.BlockSpec((CL1_F, 75), lambda i: (0, 0)),
            pl.BlockSpec((1, CL1_F), lambda i: (0, 0)),
        ],
        out_specs=pl.BlockSpec((1, IN_V, CL1_F), lambda i: (i, 0, 0)),
        out_shape=jax.ShapeDtypeStruct((B, IN_V, CL1_F), jnp.float32),
    )(x, wm, b)


def _sc_gather(table, idx):
    info = plsc.get_sparse_core_info()
    nw = info.num_cores * info.num_subcores
    b_per_w = IN_V // nw
    mesh = plsc.VectorSubcoreMesh(core_axis_name="c", subcore_axis_name="s")

    @functools.partial(
        pl.kernel, mesh=mesh,
        out_type=jax.ShapeDtypeStruct((IN_V, B * CL1_F), jnp.float32),
        scratch_types=[
            pltpu.VMEM((b_per_w,), jnp.int32),
            pltpu.VMEM((b_per_w, B * CL1_F), jnp.float32),
            pltpu.SemaphoreType.DMA,
        ],
    )
    def k(table_hbm, idx_hbm, out_hbm, idx_v, rows_v, sem):
        wid = lax.axis_index("s") * info.num_cores + lax.axis_index("c")
        base = wid * b_per_w
        pltpu.sync_copy(idx_hbm.at[pl.ds(base, b_per_w)], idx_v)
        pltpu.async_copy(table_hbm.at[idx_v], rows_v, sem).wait()
        pltpu.sync_copy(rows_v, out_hbm.at[pl.ds(base, b_per_w)])

    return k(table, idx)


def _bn_body(x_ref, s_ref, out_ref):
    x = x_ref[...]
    s = s_ref[...]
    n = float(IN_V * B)
    csum = jnp.sum(x, axis=0, keepdims=True)
    csum2 = jnp.sum(x * x, axis=0, keepdims=True)
    chs = lax.dot_general(csum, s, (((1,), (0,)), ((), ())),
                          preferred_element_type=jnp.float32)
    chs2 = lax.dot_general(csum2, s, (((1,), (0,)), ((), ())),
                           preferred_element_type=jnp.float32)
    mean = chs / n
    var = chs2 / n - mean * mean
    inv = lax.rsqrt(var + 1e-5)
    mean_c = lax.dot_general(mean, s, (((1,), (1,)), ((), ())),
                             preferred_element_type=jnp.float32)
    inv_c = lax.dot_general(inv, s, (((1,), (1,)), ((), ())),
                            preferred_element_type=jnp.float32)
    out_ref[...] = (x - mean_c) * inv_c


def _bn(x, s):
    return pl.pallas_call(
        _bn_body,
        out_shape=jax.ShapeDtypeStruct(x.shape, jnp.float32),
    )(x, s)


def _cheb1_body(l_ref, xin_ref, out_ref, lb_ref, xb_ref):
    @pl.when(pl.program_id(0) == 0)
    def _():
        xb_ref[...] = xin_ref[...].astype(jnp.bfloat16)

    lb = l_ref[...].astype(jnp.bfloat16)
    lb_ref[...] = lb
    out_ref[...] = jnp.dot(lb, xb_ref[...], preferred_element_type=jnp.float32)


def _cheb1(l_mat, xin, tile):
    v, c = l_mat.shape[0], xin.shape[1]
    return pl.pallas_call(
        _cheb1_body,
        grid=(v // tile,),
        in_specs=[
            pl.BlockSpec((tile, v), lambda i: (i, 0)),
            pl.BlockSpec((v, c), lambda i: (0, 0)),
        ],
        out_specs=[
            pl.BlockSpec((tile, c), lambda i: (i, 0)),
            pl.BlockSpec((tile, v), lambda i: (i, 0)),
        ],
        out_shape=[
            jax.ShapeDtypeStruct((v, c), jnp.float32),
            jax.ShapeDtypeStruct((v, v), jnp.bfloat16),
        ],
        scratch_shapes=[pltpu.VMEM((v, c), jnp.bfloat16)],
    )(l_mat, xin)


def _cheb_body(lb_ref, xin_ref, xprev_ref, out_ref, xb_ref):
    @pl.when(pl.program_id(0) == 0)
    def _():
        xb_ref[...] = xin_ref[...].astype(jnp.bfloat16)

    acc = jnp.dot(lb_ref[...], xb_ref[...], preferred_element_type=jnp.float32)
    out_ref[...] = 2.0 * acc - xprev_ref[...]


def _cheb(lb, xin, xprev, tile):
    v, c = lb.shape[0], xin.shape[1]
    return pl.pallas_call(
        _cheb_body,
        grid=(v // tile,),
        in_specs=[
            pl.BlockSpec((tile, v), lambda i: (i, 0)),
            pl.BlockSpec((v, c), lambda i: (0, 0)),
            pl.BlockSpec((tile, c), lambda i: (i, 0)),
        ],
        out_specs=pl.BlockSpec((tile, c), lambda i: (i, 0)),
        out_shape=jax.ShapeDtypeStruct((v, c), jnp.float32),
        scratch_shapes=[pltpu.VMEM((v, c), jnp.bfloat16)],
    )(lb, xin, xprev)


def _comb_body(nb, cin, cout, x0_ref, x1_ref, x2_ref, x3_ref, w_ref, b_ref, out_ref):
    w = w_ref[...]
    bias = b_ref[...]
    v = x0_ref.shape[0]
    outs = []
    for j in range(nb):
        sl = slice(j * cin, (j + 1) * cin)
        y = lax.dot_general(x0_ref[:, sl], w[0], (((1,), (0,)), ((), ())),
                            preferred_element_type=jnp.float32)
        y += lax.dot_general(x1_ref[:, sl], w[1], (((1,), (0,)), ((), ())),
                             preferred_element_type=jnp.float32)
        y += lax.dot_general(x2_ref[:, sl], w[2], (((1,), (0,)), ((), ())),
                             preferred_element_type=jnp.float32)
        y += lax.dot_general(x3_ref[:, sl], w[3], (((1,), (0,)), ((), ())),
                             preferred_element_type=jnp.float32)
        y = jnp.maximum(y + bias, 0.0)
        y = y.reshape(v // 4, 4, cout).max(axis=1)
        outs.append(y)
    out_ref[...] = jnp.concatenate(outs, axis=1)


def _combine(x0, x1, x2, x3, w, b, nb, cin, cout):
    v = x0.shape[0]
    grid = B // nb
    blk_in = nb * cin
    blk_out = nb * cout
    return pl.pallas_call(
        functools.partial(_comb_body, nb, cin, cout),
        grid=(grid,),
        in_specs=[
            pl.BlockSpec((v, blk_in), lambda i: (0, i)),
            pl.BlockSpec((v, blk_in), lambda i: (0, i)),
            pl.BlockSpec((v, blk_in), lambda i: (0, i)),
            pl.BlockSpec((v, blk_in), lambda i: (0, i)),
            pl.BlockSpec((K, cin, cout), lambda i: (0, 0, 0)),
            pl.BlockSpec((1, cout), lambda i: (0, 0)),
        ],
        out_specs=pl.BlockSpec((v // 4, blk_out), lambda i: (0, i)),
        out_shape=jax.ShapeDtypeStruct((v // 4, B * cout), jnp.float32),
    )(x0, x1, x2, x3, w, b)


def _comb2_body(x0_ref, x1_ref, x2_ref, x3_ref, w_ref, b_ref, out_ref):
    w = w_ref[...]
    bias = b_ref[...]
    for j in range(2):
        sl = slice(j * CL2_F, (j + 1) * CL2_F)
        y = lax.dot_general(x0_ref[:, sl], w[0], (((1,), (0,)), ((), ())),
                            preferred_element_type=jnp.float32)
        y += lax.dot_general(x1_ref[:, sl], w[1], (((1,), (0,)), ((), ())),
                             preferred_element_type=jnp.float32)
        y += lax.dot_general(x2_ref[:, sl], w[2], (((1,), (0,)), ((), ())),
                             preferred_element_type=jnp.float32)
        y += lax.dot_general(x3_ref[:, sl], w[3], (((1,), (0,)), ((), ())),
                             preferred_element_type=jnp.float32)
        y = jnp.maximum(y + bias, 0.0)
        y = y.reshape(V3, 4, CL3_F).max(axis=1)
        out_ref[j] = y.T


def _combine2(x0, x1, x2, x3, w, b):
    return pl.pallas_call(
        _comb2_body,
        grid=(8,),
        in_specs=[
            pl.BlockSpec((V2, 2 * CL2_F), lambda i: (0, i)),
            pl.BlockSpec((V2, 2 * CL2_F), lambda i: (0, i)),
            pl.BlockSpec((V2, 2 * CL2_F), lambda i: (0, i)),
            pl.BlockSpec((V2, 2 * CL2_F), lambda i: (0, i)),
            pl.BlockSpec((K, CL2_F, CL3_F), lambda i: (0, 0, 0)),
            pl.BlockSpec((1, CL3_F), lambda i: (0, 0)),
        ],
        out_specs=pl.BlockSpec((2, CL3_F, V3), lambda i: (i, 0, 0)),
        out_shape=jax.ShapeDtypeStruct((B, CL3_F, V3), jnp.float32),
    )(x0, x1, x2, x3, w, b)


def _fc_body(h_ref, w1_ref, b1_ref, w2_ref, b2_ref, out_ref, acc_ref):
    i = pl.program_id(0)

    @pl.when(i == 0)
    def _():
        acc_ref[...] = jnp.zeros_like(acc_ref)

    acc_ref[...] += lax.dot_general(h_ref[...], w1_ref[...],
                                    (((1,), (1,)), ((), ())),
                                    preferred_element_type=jnp.float32)

    @pl.when(i == pl.num_programs(0) - 1)
    def _():
        a = jnp.maximum(acc_ref[...] + b1_ref[...], 0.0)
        out_ref[...] = lax.dot_general(a, w2_ref[...], (((1,), (1,)), ((), ())),
                                       preferred_element_type=jnp.float32) + b2_ref[...]


def _fc(h, w1, b1, w2, b2):
    kchunk = 4096
    grid = FC1_IN // kchunk
    return pl.pallas_call(
        _fc_body,
        grid=(grid,),
        in_specs=[
            pl.BlockSpec((B, kchunk), lambda i: (0, i)),
            pl.BlockSpec((FC1_F, kchunk), lambda i: (0, i)),
            pl.BlockSpec((1, FC1_F), lambda i: (0, 0)),
            pl.BlockSpec((FC2_F, FC1_F), lambda i: (0, 0)),
            pl.BlockSpec((1, FC2_F), lambda i: (0, 0)),
        ],
        out_specs=pl.BlockSpec((B, FC2_F), lambda i: (0, 0)),
        out_shape=jax.ShapeDtypeStruct((B, FC2_F), jnp.float32),
        scratch_shapes=[pltpu.VMEM((B, FC1_F), jnp.float32)],
    )(h, w1, b1, w2, b2)


def kernel(x, conv1_w, conv1_b, w2, b2, w3, b3, fc1_w, fc1_b, fc2_w, fc2_b,
           node_index, perm, L0, L2):
    wm = conv1_w.transpose(0, 2, 3, 1).reshape(CL1_F, 75)
    idxflat = (node_index[:, 0] * GRID + node_index[:, 1]).astype(jnp.int32)
    perm = perm.astype(jnp.int32)
    src = jnp.where(perm < V_SEL,
                    idxflat[jnp.clip(perm, 0, V_SEL - 1)],
                    jnp.int32(IN_V)).astype(jnp.int32)
    sel = (jnp.arange(B * CL1_F, dtype=jnp.int32) % CL1_F)[:, None] == \
        jnp.arange(CL1_F, dtype=jnp.int32)[None, :]
    sel = sel.astype(jnp.float32)
    w2s = w2.reshape(CL2_F, CL1_F, K).transpose(2, 1, 0)
    w3s = w3.reshape(CL3_F, CL2_F, K).transpose(2, 1, 0)

    pooled = _conv_pool(x, wm, conv1_b.reshape(1, CL1_F))
    table = pooled.transpose(1, 0, 2).reshape(IN_V, B * CL1_F)
    table = jnp.concatenate(
        [table, jnp.zeros((VPAD - IN_V, B * CL1_F), jnp.float32)], axis=0)
    x0 = _sc_gather(table, src)
    x0 = _bn(x0, sel)
    x1, l0b = _cheb1(L0, x0, 512)
    x2 = _cheb(l0b, x1, x0, 512)
    x3 = _cheb(l0b, x2, x1, 512)
    g2 = _combine(x0, x1, x2, x3, w2s, b2.reshape(1, CL2_F),
                  4, CL1_F, CL2_F)
    y1, l2b = _cheb1(L2, g2, 512)
    y2 = _cheb(l2b, y1, g2, 512)
    y3 = _cheb(l2b, y2, y1, 512)
    h3 = _combine2(g2, y1, y2, y3, w3s, b3.reshape(1, CL3_F))
    hf = h3.reshape(B, FC1_IN)
    return _fc(hf, fc1_w, fc1_b.reshape(1, FC1_F),
               fc2_w, fc2_b.reshape(1, FC2_F))


# speedup vs baseline: 1.5345x; 1.2146x over previous
import functools
import jax
import jax.numpy as jnp
from jax import lax
from jax.experimental import pallas as pl
from jax.experimental.pallas import tpu as pltpu
from jax.experimental.pallas import tpu_sc as plsc

B = 16
CL1_F = 32
CL2_F = 64
CL3_F = 128
IN_V = 4096
V_SEL = 4000
GRID = 64
V2 = 1024
V3 = 256
K = 4
FC1_IN = CL3_F * V3
FC1_F = 512
FC2_F = 10
VPAD = 4104


def _conv_body(x_ref, wm_ref, b_ref, out_ref):
    wm = wm_ref[...]
    bias = b_ref[...]
    xflat = x_ref[0].reshape(3, 128 * 128)
    jmod = lax.broadcasted_iota(jnp.int32, (1, 128 * 128), 1) % 128
    xdx = []
    for d in range(-2, 3):
        if d > 0:
            xs = jnp.concatenate(
                [xflat[:, d:], jnp.zeros((3, d), jnp.float32)], axis=1)
            m = jmod < (128 - d)
        elif d < 0:
            xs = jnp.concatenate(
                [jnp.zeros((3, -d), jnp.float32), xflat[:, :128 * 128 + d]], axis=1)
            m = jmod >= (-d)
        else:
            xdx.append(xflat)
            continue
        xdx.append(jnp.where(m, xs, 0.0))
    taps = []
    for dy in range(5):
        e = (dy - 2) * 128
        for xd in xdx:
            if e > 0:
                taps.append(jnp.concatenate(
                    [xd[:, e:], jnp.zeros((3, e), jnp.float32)], axis=1))
            elif e < 0:
                taps.append(jnp.concatenate(
                    [jnp.zeros((3, -e), jnp.float32), xd[:, :128 * 128 + e]], axis=1))
            else:
                taps.append(xd)
    p = jnp.concatenate(taps, axis=0)
    h = lax.dot_general(p, wm, (((0,), (1,)), ((), ())),
                        preferred_element_type=jnp.float32)
    h = jnp.maximum(h + bias, 0.0)
    h = h.reshape(8192, 2, CL1_F).max(axis=1)
    h = h.reshape(64, 128, CL1_F)
    h = jnp.maximum(h[:, :64, :], h[:, 64:, :])
    out_ref[0] = h.reshape(IN_V, CL1_F)


def _conv_pool(x, wm, b):
    return pl.pallas_call(
        _conv_body,
        grid=(B,),
        in_specs=[
            pl.BlockSpec((1, 3, 128, 128), lambda i: (i, 0, 0, 0)),
            pl.BlockSpec((CL1_F, 75), lambda i: (0, 0)),
            pl.BlockSpec((1, CL1_F), lambda i: (0, 0)),
        ],
        out_specs=pl.BlockSpec((1, IN_V, CL1_F), lambda i: (i, 0, 0)),
        out_shape=jax.ShapeDtypeStruct((B, IN_V, CL1_F), jnp.float32),
    )(x, wm, b)


def _sc_gather(table, idx):
    info = plsc.get_sparse_core_info()
    nw = info.num_cores * info.num_subcores
    b_per_w = IN_V // nw
    mesh = plsc.VectorSubcoreMesh(core_axis_name="c", subcore_axis_name="s")

    @functools.partial(
        pl.kernel, mesh=mesh,
        out_type=jax.ShapeDtypeStruct((IN_V, B * CL1_F), jnp.float32),
        scratch_types=[
            pltpu.VMEM((b_per_w,), jnp.int32),
            pltpu.VMEM((b_per_w, B * CL1_F), jnp.float32),
            pltpu.SemaphoreType.DMA,
        ],
    )
    def k(table_hbm, idx_hbm, out_hbm, idx_v, rows_v, sem):
        wid = lax.axis_index("s") * info.num_cores + lax.axis_index("c")
        base = wid * b_per_w
        pltpu.sync_copy(idx_hbm.at[pl.ds(base, b_per_w)], idx_v)
        pltpu.async_copy(table_hbm.at[idx_v], rows_v, sem).wait()
        pltpu.sync_copy(rows_v, out_hbm.at[pl.ds(base, b_per_w)])

    return k(table, idx)


def _bn_body(x_ref, s_ref, out_ref):
    x = x_ref[...]
    s = s_ref[...]
    n = float(IN_V * B)
    csum = jnp.sum(x, axis=0, keepdims=True)
    csum2 = jnp.sum(x * x, axis=0, keepdims=True)
    chs = lax.dot_general(csum, s, (((1,), (0,)), ((), ())),
                          preferred_element_type=jnp.float32)
    chs2 = lax.dot_general(csum2, s, (((1,), (0,)), ((), ())),
                           preferred_element_type=jnp.float32)
    mean = chs / n
    var = chs2 / n - mean * mean
    inv = lax.rsqrt(var + 1e-5)
    mean_c = lax.dot_general(mean, s, (((1,), (1,)), ((), ())),
                             preferred_element_type=jnp.float32)
    inv_c = lax.dot_general(inv, s, (((1,), (1,)), ((), ())),
                            preferred_element_type=jnp.float32)
    out_ref[...] = (x - mean_c) * inv_c


def _bn(x, s):
    return pl.pallas_call(
        _bn_body,
        out_shape=jax.ShapeDtypeStruct(x.shape, jnp.float32),
    )(x, s)


def _cheb1_body(l_ref, xin_ref, out_ref, lb_ref, xb_ref):
    @pl.when(pl.program_id(0) == 0)
    def _():
        xb_ref[...] = xin_ref[...].astype(jnp.bfloat16)

    lb = l_ref[...].astype(jnp.bfloat16)
    lb_ref[...] = lb
    out_ref[...] = jnp.dot(lb, xb_ref[...], preferred_element_type=jnp.float32)


def _cheb1(l_mat, xin, tile):
    v, c = l_mat.shape[0], xin.shape[1]
    return pl.pallas_call(
        _cheb1_body,
        grid=(v // tile,),
        in_specs=[
            pl.BlockSpec((tile, v), lambda i: (i, 0)),
            pl.BlockSpec((v, c), lambda i: (0, 0)),
        ],
        out_specs=[
            pl.BlockSpec((tile, c), lambda i: (i, 0)),
            pl.BlockSpec((tile, v), lambda i: (i, 0)),
        ],
        out_shape=[
            jax.ShapeDtypeStruct((v, c), jnp.float32),
            jax.ShapeDtypeStruct((v, v), jnp.bfloat16),
        ],
        scratch_shapes=[pltpu.VMEM((v, c), jnp.bfloat16)],
    )(l_mat, xin)


def _cheb_body(lb_ref, xin_ref, xprev_ref, out_ref, xb_ref):
    @pl.when(pl.program_id(0) == 0)
    def _():
        xb_ref[...] = xin_ref[...].astype(jnp.bfloat16)

    acc = jnp.dot(lb_ref[...], xb_ref[...], preferred_element_type=jnp.float32)
    out_ref[...] = 2.0 * acc - xprev_ref[...]


def _cheb(lb, xin, xprev, tile):
    v, c = lb.shape[0], xin.shape[1]
    return pl.pallas_call(
        _cheb_body,
        grid=(v // tile,),
        in_specs=[
            pl.BlockSpec((tile, v), lambda i: (i, 0)),
            pl.BlockSpec((v, c), lambda i: (0, 0)),
            pl.BlockSpec((tile, c), lambda i: (i, 0)),
        ],
        out_specs=pl.BlockSpec((tile, c), lambda i: (i, 0)),
        out_shape=jax.ShapeDtypeStruct((v, c), jnp.float32),
        scratch_shapes=[pltpu.VMEM((v, c), jnp.bfloat16)],
    )(lb, xin, xprev)


def _comb_body(nb, cin, cout, x0_ref, x1_ref, x2_ref, x3_ref, w_ref, b_ref, out_ref):
    w = w_ref[...]
    bias = b_ref[...]
    v = x0_ref.shape[0]
    outs = []
    for j in range(nb):
        sl = slice(j * cin, (j + 1) * cin)
        y = lax.dot_general(x0_ref[:, sl], w[0], (((1,), (0,)), ((), ())),
                            preferred_element_type=jnp.float32)
        y += lax.dot_general(x1_ref[:, sl], w[1], (((1,), (0,)), ((), ())),
                             preferred_element_type=jnp.float32)
        y += lax.dot_general(x2_ref[:, sl], w[2], (((1,), (0,)), ((), ())),
                             preferred_element_type=jnp.float32)
        y += lax.dot_general(x3_ref[:, sl], w[3], (((1,), (0,)), ((), ())),
                             preferred_element_type=jnp.float32)
        y = jnp.maximum(y + bias, 0.0)
        y = y.reshape(v // 4, 4, cout).max(axis=1)
        outs.append(y)
    out_ref[...] = jnp.concatenate(outs, axis=1)


def _combine(x0, x1, x2, x3, w, b, nb, cin, cout):
    v = x0.shape[0]
    grid = B // nb
    blk_in = nb * cin
    blk_out = nb * cout
    return pl.pallas_call(
        functools.partial(_comb_body, nb, cin, cout),
        grid=(grid,),
        in_specs=[
            pl.BlockSpec((v, blk_in), lambda i: (0, i)),
            pl.BlockSpec((v, blk_in), lambda i: (0, i)),
            pl.BlockSpec((v, blk_in), lambda i: (0, i)),
            pl.BlockSpec((v, blk_in), lambda i: (0, i)),
            pl.BlockSpec((K, cin, cout), lambda i: (0, 0, 0)),
            pl.BlockSpec((1, cout), lambda i: (0, 0)),
        ],
        out_specs=pl.BlockSpec((v // 4, blk_out), lambda i: (0, i)),
        out_shape=jax.ShapeDtypeStruct((v // 4, B * cout), jnp.float32),
    )(x0, x1, x2, x3, w, b)


def _comb2_body(x0_ref, x1_ref, x2_ref, x3_ref, w_ref, b_ref, out_ref):
    w = w_ref[...]
    bias = b_ref[...]
    for j in range(2):
        sl = slice(j * CL2_F, (j + 1) * CL2_F)
        y = lax.dot_general(x0_ref[:, sl], w[0], (((1,), (0,)), ((), ())),
                            preferred_element_type=jnp.float32)
        y += lax.dot_general(x1_ref[:, sl], w[1], (((1,), (0,)), ((), ())),
                             preferred_element_type=jnp.float32)
        y += lax.dot_general(x2_ref[:, sl], w[2], (((1,), (0,)), ((), ())),
                             preferred_element_type=jnp.float32)
        y += lax.dot_general(x3_ref[:, sl], w[3], (((1,), (0,)), ((), ())),
                             preferred_element_type=jnp.float32)
        y = jnp.maximum(y + bias, 0.0)
        y = y.reshape(V3, 4, CL3_F).max(axis=1)
        out_ref[j] = y.T


def _combine2(x0, x1, x2, x3, w, b):
    return pl.pallas_call(
        _comb2_body,
        grid=(8,),
        in_specs=[
            pl.BlockSpec((V2, 2 * CL2_F), lambda i: (0, i)),
            pl.BlockSpec((V2, 2 * CL2_F), lambda i: (0, i)),
            pl.BlockSpec((V2, 2 * CL2_F), lambda i: (0, i)),
            pl.BlockSpec((V2, 2 * CL2_F), lambda i: (0, i)),
            pl.BlockSpec((K, CL2_F, CL3_F), lambda i: (0, 0, 0)),
            pl.BlockSpec((1, CL3_F), lambda i: (0, 0)),
        ],
        out_specs=pl.BlockSpec((2, CL3_F, V3), lambda i: (i, 0, 0)),
        out_shape=jax.ShapeDtypeStruct((B, CL3_F, V3), jnp.float32),
    )(x0, x1, x2, x3, w, b)


def _fc_body(h_ref, w1_ref, b1_ref, w2_ref, b2_ref, out_ref, acc_ref):
    i = pl.program_id(0)

    @pl.when(i == 0)
    def _():
        acc_ref[...] = jnp.zeros_like(acc_ref)

    acc_ref[...] += lax.dot_general(h_ref[...], w1_ref[...],
                                    (((1,), (1,)), ((), ())),
                                    preferred_element_type=jnp.float32)

    @pl.when(i == pl.num_programs(0) - 1)
    def _():
        a = jnp.maximum(acc_ref[...] + b1_ref[...], 0.0)
        out_ref[...] = lax.dot_general(a, w2_ref[...], (((1,), (1,)), ((), ())),
                                       preferred_element_type=jnp.float32) + b2_ref[...]


def _fc(h, w1, b1, w2, b2):
    kchunk = 4096
    grid = FC1_IN // kchunk
    return pl.pallas_call(
        _fc_body,
        grid=(grid,),
        in_specs=[
            pl.BlockSpec((B, kchunk), lambda i: (0, i)),
            pl.BlockSpec((FC1_F, kchunk), lambda i: (0, i)),
            pl.BlockSpec((1, FC1_F), lambda i: (0, 0)),
            pl.BlockSpec((FC2_F, FC1_F), lambda i: (0, 0)),
            pl.BlockSpec((1, FC2_F), lambda i: (0, 0)),
        ],
        out_specs=pl.BlockSpec((B, FC2_F), lambda i: (0, 0)),
        out_shape=jax.ShapeDtypeStruct((B, FC2_F), jnp.float32),
        scratch_shapes=[pltpu.VMEM((B, FC1_F), jnp.float32)],
    )(h, w1, b1, w2, b2)


def kernel(x, conv1_w, conv1_b, w2, b2, w3, b3, fc1_w, fc1_b, fc2_w, fc2_b,
           node_index, perm, L0, L2):
    wm = conv1_w.transpose(0, 2, 3, 1).reshape(CL1_F, 75)
    idxflat = (node_index[:, 0] * GRID + node_index[:, 1]).astype(jnp.int32)
    perm = perm.astype(jnp.int32)
    src = jnp.where(perm < V_SEL,
                    idxflat[jnp.clip(perm, 0, V_SEL - 1)],
                    jnp.int32(IN_V)).astype(jnp.int32)
    sel = (jnp.arange(B * CL1_F, dtype=jnp.int32) % CL1_F)[:, None] == \
        jnp.arange(CL1_F, dtype=jnp.int32)[None, :]
    sel = sel.astype(jnp.float32)
    w2s = w2.reshape(CL2_F, CL1_F, K).transpose(2, 1, 0)
    w3s = w3.reshape(CL3_F, CL2_F, K).transpose(2, 1, 0)

    pooled = _conv_pool(x, wm, conv1_b.reshape(1, CL1_F))
    table = pooled.transpose(1, 0, 2).reshape(IN_V, B * CL1_F)
    table = jnp.concatenate(
        [table, jnp.zeros((VPAD - IN_V, B * CL1_F), jnp.float32)], axis=0)
    x0 = _sc_gather(table, src)
    x0 = _bn(x0, sel)
    x1, l0b = _cheb1(L0, x0, 512)
    x2 = _cheb(l0b, x1, x0, 512)
    x3 = _cheb(l0b, x2, x1, 512)
    g2 = _combine(x0, x1, x2, x3, w2s, b2.reshape(1, CL2_F),
                  4, CL1_F, CL2_F)
    y1, l2b = _cheb1(L2, g2, 512)
    y2 = _cheb(l2b, y1, g2, 512)
    y3 = _cheb(l2b, y2, y1, 512)
    h3 = _combine2(g2, y1, y2, y3, w3s, b3.reshape(1, CL3_F))
    hf = h3.reshape(B, FC1_IN)
    return _fc(hf, fc1_w, fc1_b.reshape(1, FC1_F),
               fc2_w, fc2_b.reshape(1, FC2_F))


# speedup vs baseline: 1.9016x; 1.2392x over previous
import functools
import jax
import jax.numpy as jnp
from jax import lax
from jax.experimental import pallas as pl
from jax.experimental.pallas import tpu as pltpu
from jax.experimental.pallas import tpu_sc as plsc

B = 16
CL1_F = 32
CL2_F = 64
CL3_F = 128
IN_V = 4096
V_SEL = 4000
GRID = 64
V2 = 1024
V3 = 256
K = 4
FC1_IN = CL3_F * V3
FC1_F = 512
FC2_F = 10
VPAD = 4104


def _conv_body(x_ref, wm_ref, b_ref, out_ref):
    wm = wm_ref[...]
    bias = b_ref[...]
    jmod = lax.broadcasted_iota(jnp.int32, (1, 64 * 64), 1) % 64
    shifted = {}
    for u in range(-2, 4):
        ap, si = u % 2, u // 2
        for v in range(-2, 4):
            bp, sj = v % 2, v // 2
            f = x_ref[0, 2 * ap + bp].reshape(3, 64 * 64)
            e = si * 64 + sj
            if e > 0:
                f = jnp.concatenate(
                    [f[:, e:], jnp.zeros((3, e), jnp.float32)], axis=1)
            elif e < 0:
                f = jnp.concatenate(
                    [jnp.zeros((3, -e), jnp.float32), f[:, :64 * 64 + e]], axis=1)
            if sj > 0:
                f = jnp.where(jmod < 64 - sj, f, 0.0)
            elif sj < 0:
                f = jnp.where(jmod >= -sj, f, 0.0)
            shifted[(u, v)] = f
    g = None
    for a in range(2):
        for b in range(2):
            taps = []
            for dy in range(5):
                for dx in range(5):
                    taps.append(shifted[(a + dy - 2, b + dx - 2)])
            p = jnp.concatenate(taps, axis=0)
            gab = lax.dot_general(p, wm, (((0,), (1,)), ((), ())),
                                  preferred_element_type=jnp.float32)
            g = gab if g is None else jnp.maximum(g, gab)
    out_ref[0] = jnp.maximum(g + bias, 0.0)


def _conv_pool(x, wm, b):
    return pl.pallas_call(
        _conv_body,
        grid=(B,),
        in_specs=[
            pl.BlockSpec((1, 4, 3, 64, 64), lambda i: (i, 0, 0, 0, 0)),
            pl.BlockSpec((CL1_F, 75), lambda i: (0, 0)),
            pl.BlockSpec((1, CL1_F), lambda i: (0, 0)),
        ],
        out_specs=pl.BlockSpec((1, IN_V, CL1_F), lambda i: (i, 0, 0)),
        out_shape=jax.ShapeDtypeStruct((B, IN_V, CL1_F), jnp.float32),
    )(x, wm, b)


def _sc_gather(table, idx):
    info = plsc.get_sparse_core_info()
    nw = info.num_cores * info.num_subcores
    b_per_w = IN_V // nw
    mesh = plsc.VectorSubcoreMesh(core_axis_name="c", subcore_axis_name="s")

    @functools.partial(
        pl.kernel, mesh=mesh,
        out_type=jax.ShapeDtypeStruct((IN_V, B * CL1_F), jnp.float32),
        scratch_types=[
            pltpu.VMEM((b_per_w,), jnp.int32),
            pltpu.VMEM((b_per_w, B * CL1_F), jnp.float32),
            pltpu.SemaphoreType.DMA,
        ],
    )
    def k(table_hbm, idx_hbm, out_hbm, idx_v, rows_v, sem):
        wid = lax.axis_index("s") * info.num_cores + lax.axis_index("c")
        base = wid * b_per_w
        pltpu.sync_copy(idx_hbm.at[pl.ds(base, b_per_w)], idx_v)
        pltpu.async_copy(table_hbm.at[idx_v], rows_v, sem).wait()
        pltpu.sync_copy(rows_v, out_hbm.at[pl.ds(base, b_per_w)])

    return k(table, idx)


def _bn_body(x_ref, s_ref, out_ref):
    x = x_ref[...]
    s = s_ref[...]
    n = float(IN_V * B)
    csum = jnp.sum(x, axis=0, keepdims=True)
    csum2 = jnp.sum(x * x, axis=0, keepdims=True)
    chs = lax.dot_general(csum, s, (((1,), (0,)), ((), ())),
                          preferred_element_type=jnp.float32)
    chs2 = lax.dot_general(csum2, s, (((1,), (0,)), ((), ())),
                           preferred_element_type=jnp.float32)
    mean = chs / n
    var = chs2 / n - mean * mean
    inv = lax.rsqrt(var + 1e-5)
    mean_c = lax.dot_general(mean, s, (((1,), (1,)), ((), ())),
                             preferred_element_type=jnp.float32)
    inv_c = lax.dot_general(inv, s, (((1,), (1,)), ((), ())),
                            preferred_element_type=jnp.float32)
    out_ref[...] = (x - mean_c) * inv_c


def _bn(x, s):
    return pl.pallas_call(
        _bn_body,
        out_shape=jax.ShapeDtypeStruct(x.shape, jnp.float32),
    )(x, s)


def _cheb1_body(l_ref, xin_ref, out_ref, lb_ref, xb_ref):
    @pl.when(pl.program_id(0) == 0)
    def _():
        xb_ref[...] = xin_ref[...].astype(jnp.bfloat16)

    lb = l_ref[...].astype(jnp.bfloat16)
    lb_ref[...] = lb
    out_ref[...] = jnp.dot(lb, xb_ref[...], preferred_element_type=jnp.float32)


def _cheb1(l_mat, xin, tile):
    v, c = l_mat.shape[0], xin.shape[1]
    return pl.pallas_call(
        _cheb1_body,
        grid=(v // tile,),
        in_specs=[
            pl.BlockSpec((tile, v), lambda i: (i, 0)),
            pl.BlockSpec((v, c), lambda i: (0, 0)),
        ],
        out_specs=[
            pl.BlockSpec((tile, c), lambda i: (i, 0)),
            pl.BlockSpec((tile, v), lambda i: (i, 0)),
        ],
        out_shape=[
            jax.ShapeDtypeStruct((v, c), jnp.float32),
            jax.ShapeDtypeStruct((v, v), jnp.bfloat16),
        ],
        scratch_shapes=[pltpu.VMEM((v, c), jnp.bfloat16)],
    )(l_mat, xin)


def _cheb_body(lb_ref, xin_ref, xprev_ref, out_ref, xb_ref):
    @pl.when(pl.program_id(0) == 0)
    def _():
        xb_ref[...] = xin_ref[...].astype(jnp.bfloat16)

    acc = jnp.dot(lb_ref[...], xb_ref[...], preferred_element_type=jnp.float32)
    out_ref[...] = 2.0 * acc - xprev_ref[...]


def _cheb(lb, xin, xprev, tile):
    v, c = lb.shape[0], xin.shape[1]
    return pl.pallas_call(
        _cheb_body,
        grid=(v // tile,),
        in_specs=[
            pl.BlockSpec((tile, v), lambda i: (i, 0)),
            pl.BlockSpec((v, c), lambda i: (0, 0)),
            pl.BlockSpec((tile, c), lambda i: (i, 0)),
        ],
        out_specs=pl.BlockSpec((tile, c), lambda i: (i, 0)),
        out_shape=jax.ShapeDtypeStruct((v, c), jnp.float32),
        scratch_shapes=[pltpu.VMEM((v, c), jnp.bfloat16)],
    )(lb, xin, xprev)


def _comb_body(nb, cin, cout, x0_ref, x1_ref, x2_ref, x3_ref, w_ref, b_ref, out_ref):
    w = w_ref[...]
    bias = b_ref[...]
    v = x0_ref.shape[0]
    outs = []
    for j in range(nb):
        sl = slice(j * cin, (j + 1) * cin)
        y = lax.dot_general(x0_ref[:, sl], w[0], (((1,), (0,)), ((), ())),
                            preferred_element_type=jnp.float32)
        y += lax.dot_general(x1_ref[:, sl], w[1], (((1,), (0,)), ((), ())),
                             preferred_element_type=jnp.float32)
        y += lax.dot_general(x2_ref[:, sl], w[2], (((1,), (0,)), ((), ())),
                             preferred_element_type=jnp.float32)
        y += lax.dot_general(x3_ref[:, sl], w[3], (((1,), (0,)), ((), ())),
                             preferred_element_type=jnp.float32)
        y = jnp.maximum(y + bias, 0.0)
        y = y.reshape(v // 4, 4, cout).max(axis=1)
        outs.append(y)
    out_ref[...] = jnp.concatenate(outs, axis=1)


def _combine(x0, x1, x2, x3, w, b, nb, cin, cout):
    v = x0.shape[0]
    grid = B // nb
    blk_in = nb * cin
    blk_out = nb * cout
    return pl.pallas_call(
        functools.partial(_comb_body, nb, cin, cout),
        grid=(grid,),
        in_specs=[
            pl.BlockSpec((v, blk_in), lambda i: (0, i)),
            pl.BlockSpec((v, blk_in), lambda i: (0, i)),
            pl.BlockSpec((v, blk_in), lambda i: (0, i)),
            pl.BlockSpec((v, blk_in), lambda i: (0, i)),
            pl.BlockSpec((K, cin, cout), lambda i: (0, 0, 0)),
            pl.BlockSpec((1, cout), lambda i: (0, 0)),
        ],
        out_specs=pl.BlockSpec((v // 4, blk_out), lambda i: (0, i)),
        out_shape=jax.ShapeDtypeStruct((v // 4, B * cout), jnp.float32),
    )(x0, x1, x2, x3, w, b)


def _comb2_body(x0_ref, x1_ref, x2_ref, x3_ref, w_ref, b_ref, out_ref):
    w = w_ref[...]
    bias = b_ref[...]
    for j in range(2):
        sl = slice(j * CL2_F, (j + 1) * CL2_F)
        y = lax.dot_general(x0_ref[:, sl], w[0], (((1,), (0,)), ((), ())),
                            preferred_element_type=jnp.float32)
        y += lax.dot_general(x1_ref[:, sl], w[1], (((1,), (0,)), ((), ())),
                             preferred_element_type=jnp.float32)
        y += lax.dot_general(x2_ref[:, sl], w[2], (((1,), (0,)), ((), ())),
                             preferred_element_type=jnp.float32)
        y += lax.dot_general(x3_ref[:, sl], w[3], (((1,), (0,)), ((), ())),
                             preferred_element_type=jnp.float32)
        y = jnp.maximum(y + bias, 0.0)
        y = y.reshape(V3, 4, CL3_F).max(axis=1)
        out_ref[j] = y.T


def _combine2(x0, x1, x2, x3, w, b):
    return pl.pallas_call(
        _comb2_body,
        grid=(8,),
        in_specs=[
            pl.BlockSpec((V2, 2 * CL2_F), lambda i: (0, i)),
            pl.BlockSpec((V2, 2 * CL2_F), lambda i: (0, i)),
            pl.BlockSpec((V2, 2 * CL2_F), lambda i: (0, i)),
            pl.BlockSpec((V2, 2 * CL2_F), lambda i: (0, i)),
            pl.BlockSpec((K, CL2_F, CL3_F), lambda i: (0, 0, 0)),
            pl.BlockSpec((1, CL3_F), lambda i: (0, 0)),
        ],
        out_specs=pl.BlockSpec((2, CL3_F, V3), lambda i: (i, 0, 0)),
        out_shape=jax.ShapeDtypeStruct((B, CL3_F, V3), jnp.float32),
    )(x0, x1, x2, x3, w, b)


def _fc_body(h_ref, w1_ref, b1_ref, w2_ref, b2_ref, out_ref, acc_ref):
    i = pl.program_id(0)

    @pl.when(i == 0)
    def _():
        acc_ref[...] = jnp.zeros_like(acc_ref)

    acc_ref[...] += lax.dot_general(h_ref[...], w1_ref[...],
                                    (((1,), (1,)), ((), ())),
                                    preferred_element_type=jnp.float32)

    @pl.when(i == pl.num_programs(0) - 1)
    def _():
        a = jnp.maximum(acc_ref[...] + b1_ref[...], 0.0)
        out_ref[...] = lax.dot_general(a, w2_ref[...], (((1,), (1,)), ((), ())),
                                       preferred_element_type=jnp.float32) + b2_ref[...]


def _fc(h, w1, b1, w2, b2):
    kchunk = 4096
    grid = FC1_IN // kchunk
    return pl.pallas_call(
        _fc_body,
        grid=(grid,),
        in_specs=[
            pl.BlockSpec((B, kchunk), lambda i: (0, i)),
            pl.BlockSpec((FC1_F, kchunk), lambda i: (0, i)),
            pl.BlockSpec((1, FC1_F), lambda i: (0, 0)),
            pl.BlockSpec((FC2_F, FC1_F), lambda i: (0, 0)),
            pl.BlockSpec((1, FC2_F), lambda i: (0, 0)),
        ],
        out_specs=pl.BlockSpec((B, FC2_F), lambda i: (0, 0)),
        out_shape=jax.ShapeDtypeStruct((B, FC2_F), jnp.float32),
        scratch_shapes=[pltpu.VMEM((B, FC1_F), jnp.float32)],
    )(h, w1, b1, w2, b2)


def kernel(x, conv1_w, conv1_b, w2, b2, w3, b3, fc1_w, fc1_b, fc2_w, fc2_b,
           node_index, perm, L0, L2):
    wm = conv1_w.transpose(0, 2, 3, 1).reshape(CL1_F, 75)
    idxflat = (node_index[:, 0] * GRID + node_index[:, 1]).astype(jnp.int32)
    perm = perm.astype(jnp.int32)
    src = jnp.where(perm < V_SEL,
                    idxflat[jnp.clip(perm, 0, V_SEL - 1)],
                    jnp.int32(IN_V)).astype(jnp.int32)
    sel = (jnp.arange(B * CL1_F, dtype=jnp.int32) % CL1_F)[:, None] == \
        jnp.arange(CL1_F, dtype=jnp.int32)[None, :]
    sel = sel.astype(jnp.float32)
    w2s = w2.reshape(CL2_F, CL1_F, K).transpose(2, 1, 0)
    w3s = w3.reshape(CL3_F, CL2_F, K).transpose(2, 1, 0)

    xq = x.reshape(B, 3, 64, 2, 64, 2).transpose(0, 3, 5, 1, 2, 4)
    xq = xq.reshape(B, 4, 3, 64, 64)
    pooled = _conv_pool(xq, wm, conv1_b.reshape(1, CL1_F))
    table = pooled.transpose(1, 0, 2).reshape(IN_V, B * CL1_F)
    table = jnp.concatenate(
        [table, jnp.zeros((VPAD - IN_V, B * CL1_F), jnp.float32)], axis=0)
    x0 = _sc_gather(table, src)
    x0 = _bn(x0, sel)
    x1, l0b = _cheb1(L0, x0, 512)
    x2 = _cheb(l0b, x1, x0, 512)
    x3 = _cheb(l0b, x2, x1, 512)
    g2 = _combine(x0, x1, x2, x3, w2s, b2.reshape(1, CL2_F),
                  4, CL1_F, CL2_F)
    y1, l2b = _cheb1(L2, g2, 512)
    y2 = _cheb(l2b, y1, g2, 512)
    y3 = _cheb(l2b, y2, y1, 512)
    h3 = _combine2(g2, y1, y2, y3, w3s, b3.reshape(1, CL3_F))
    hf = h3.reshape(B, FC1_IN)
    return _fc(hf, fc1_w, fc1_b.reshape(1, FC1_F),
               fc2_w, fc2_b.reshape(1, FC2_F))
